# baseline (speedup 1.0000x reference)
import jax
import jax.numpy as jnp
from jax.experimental import pallas as pl
from jax.experimental.pallas import tpu as pltpu


def _pick_tc(T):
    for c in (8, 4, 2, 1):
        if T % c == 0:
            return c
    return 1


def _col_scale(G, Hp):
    lane = jax.lax.broadcasted_iota(jnp.int32, (1, G), 1)
    return jnp.where(lane // Hp == 2, 1.0, 0.5).astype(jnp.float32)


def _dual_cell(th, c, Hp, G):
    i_f = 0.5 + 0.5 * th[:, 0 * Hp:1 * Hp]
    f_f = 0.5 + 0.5 * th[:, 1 * Hp:2 * Hp]
    g_f = th[:, 2 * Hp:3 * Hp]
    o_f = 0.5 + 0.5 * th[:, 3 * Hp:4 * Hp]
    i_b = 0.5 + 0.5 * th[:, G + 0 * Hp:G + 1 * Hp]
    f_b = 0.5 + 0.5 * th[:, G + 1 * Hp:G + 2 * Hp]
    g_b = th[:, G + 2 * Hp:G + 3 * Hp]
    o_b = 0.5 + 0.5 * th[:, G + 3 * Hp:G + 4 * Hp]
    cf = f_f * c[:, :Hp] + i_f * g_f
    cb = f_b * c[:, Hp:] + i_b * g_b
    hf = o_f * jnp.tanh(cf)
    hb = o_b * jnp.tanh(cb)
    return hf, hb, cf, cb


def _make_fused_kernel(Tc, B, B2, Hp, nT):
    G = 4 * Hp
    RB = Tc * B2

    def body(tok_ref, emb_ref, w0f_ref, w0b_ref, b0f_ref, b0b_ref,
             whh0f_ref, whh0b_ref,
             w1f0_ref, w1f1_ref, w1b0_ref, w1b1_ref, b1f_ref, b1b_ref,
             whh1f_ref, whh1b_ref, fcwf_ref, fcwb_ref, fcb_ref,
             out_ref,
             h_sc, c_sc, x_sc, hfseq_sc, hbseq_sc, head_sc,
             wbig0_sc, wbig1_sc, w1f_sc, w1b_sc, w0f_sc, w0b_sc):
        cc = pl.program_id(0)
        p = pl.program_id(1)
        t = pl.program_id(2)

        @pl.when((p == 0) & (t == 0))
        def _build_weights():
            bf16 = jnp.bfloat16
            lane = jax.lax.broadcasted_iota(jnp.int32, (1, G), 1)
            csc = jnp.where(lane // Hp == 2, 1.0, 0.5).astype(jnp.float32)
            wbig0_sc[...] = jnp.zeros_like(wbig0_sc)
            wbig0_sc[:Hp, :G] = (whh0f_ref[...] * csc).astype(bf16)
            wbig0_sc[Hp:, G:] = (whh0b_ref[...] * csc).astype(bf16)
            wbig1_sc[...] = jnp.zeros_like(wbig1_sc)
            wbig1_sc[:Hp, :G] = (whh1f_ref[...] * csc).astype(bf16)
            wbig1_sc[Hp:, G:] = (whh1b_ref[...] * csc).astype(bf16)
            w1f_sc[:Hp, :] = (w1f0_ref[...] * csc).astype(bf16)
            w1f_sc[Hp:, :] = (w1f1_ref[...] * csc).astype(bf16)
            w1b_sc[:Hp, :] = (w1b0_ref[...] * csc).astype(bf16)
            w1b_sc[Hp:, :] = (w1b1_ref[...] * csc).astype(bf16)
            w0f_sc[...] = (w0f_ref[...] * csc).astype(bf16)
            w0b_sc[...] = (w0b_ref[...] * csc).astype(bf16)

        @pl.when(p == 0)
        def _gather():
            for j in range(Tc):
                src_base = (t * Tc + j) * B + cc * B2
                dst_base = t * RB + j * B2
                for b in range(B2):
                    tok = tok_ref[src_base + b]
                    x_sc[pl.ds(dst_base + b, 1), :] = emb_ref[pl.ds(tok, 1), :]

        @pl.when((p == 1) | (p == 2))
        def _reinit_state():
            @pl.when(t == 0)
            def _z():
                h_sc[...] = jnp.zeros_like(h_sc)
                c_sc[...] = jnp.zeros_like(c_sc)


        @pl.when(p == 1)
        def _layer0():
            xf = x_sc[pl.ds(t * RB, RB), :].astype(jnp.bfloat16)
            xb = x_sc[pl.ds((nT - 1 - t) * RB, RB), :].astype(jnp.bfloat16)
            csc = _col_scale(G, Hp)
            pf = jnp.dot(xf, w0f_sc[...],
                         preferred_element_type=jnp.float32) + b0f_ref[...] * csc
            pb = jnp.dot(xb, w0b_sc[...],
                         preferred_element_type=jnp.float32) + b0b_ref[...] * csc
            wbig = wbig0_sc[...]
            h = h_sc[...]
            c = c_sc[...]
            for s in range(Tc):
                gd = jnp.dot(h.astype(jnp.bfloat16), wbig,
                             preferred_element_type=jnp.float32)
                pcat = jnp.concatenate(
                    [pf[s * B2:(s + 1) * B2],
                     pb[(Tc - 1 - s) * B2:(Tc - s) * B2]], axis=1)
                th = jnp.tanh(gd + pcat)
                hf, hb, cf, cb = _dual_cell(th, c, Hp, G)
                hfseq_sc[pl.ds(t * RB + s * B2, B2), :] = hf.astype(jnp.bfloat16)
                hbseq_sc[pl.ds((nT - 1 - t) * RB + (Tc - 1 - s) * B2, B2), :] = (
                    hb.astype(jnp.bfloat16))
                h = jnp.concatenate([hf, hb], axis=1)
                c = jnp.concatenate([cf, cb], axis=1)
            h_sc[...] = h
            c_sc[...] = c

        @pl.when(p == 2)
        def _layer1():
            catf = jnp.concatenate(
                [hfseq_sc[pl.ds(t * RB, RB), :],
                 hbseq_sc[pl.ds(t * RB, RB), :]], axis=1)
            catb = jnp.concatenate(
                [hfseq_sc[pl.ds((nT - 1 - t) * RB, RB), :],
                 hbseq_sc[pl.ds((nT - 1 - t) * RB, RB), :]], axis=1)
            csc = _col_scale(G, Hp)
            pf = jnp.dot(catf, w1f_sc[...],
                         preferred_element_type=jnp.float32) + b1f_ref[...] * csc
            pb = jnp.dot(catb, w1b_sc[...],
                         preferred_element_type=jnp.float32) + b1b_ref[...] * csc
            wbig = wbig1_sc[...]
            h = h_sc[...]
            c = c_sc[...]
            hb_first = None
            for s in range(Tc):
                gd = jnp.dot(h.astype(jnp.bfloat16), wbig,
                             preferred_element_type=jnp.float32)
                pcat = jnp.concatenate(
                    [pf[s * B2:(s + 1) * B2],
                     pb[(Tc - 1 - s) * B2:(Tc - s) * B2]], axis=1)
                th = jnp.tanh(gd + pcat)
                hf, hb, cf, cb = _dual_cell(th, c, Hp, G)
                if s == 0:
                    hb_first = hb
                h = jnp.concatenate([hf, hb], axis=1)
                c = jnp.concatenate([cf, cb], axis=1)
            h_sc[...] = h
            c_sc[...] = c

            @pl.when(t == 0)
            def _store_bwd_head():
                head_sc[...] = jnp.dot(
                    hb_first, fcwb_ref[...],
                    preferred_element_type=jnp.float32) + fcb_ref[...]

            @pl.when(t == nT - 1)
            def _finalize():
                logits = head_sc[...] + jnp.dot(
                    h[:, :Hp], fcwf_ref[...],
                    preferred_element_type=jnp.float32)
                m = jnp.max(logits, axis=-1, keepdims=True)
                shifted = logits - m
                lse = jnp.log(
                    jnp.sum(jnp.exp(shifted), axis=-1, keepdims=True))
                out_ref[...] = shifted - lse

    return body


def kernel(embedding, l0_w_in_f0, l0_w_in_b0, l0_b_f, l0_b_b, l0_whh_f,
           l0_whh_b, l1_w_in_f0, l1_w_in_f1, l1_w_in_b0, l1_w_in_b1, l1_b_f,
           l1_b_b, l1_whh_f, l1_whh_b, fc_wf, fc_wb, fc_b, tokens):
    T, B = tokens.shape
    V, E = embedding.shape
    Hp = l0_whh_f.shape[0]
    G = 4 * Hp
    O = fc_wf.shape[1]
    Tc = _pick_tc(T)
    nT = T // Tc
    B2 = B // 2 if B % 2 == 0 else B
    nC = 2 if B % 2 == 0 else 1

    const = lambda c, p, t, tok: (0, 0)

    out = pl.pallas_call(
        _make_fused_kernel(Tc, B, B2, Hp, nT),
        out_shape=jax.ShapeDtypeStruct((B, O), jnp.float32),
        grid_spec=pltpu.PrefetchScalarGridSpec(
            num_scalar_prefetch=1,
            grid=(nC, 3, nT),
            in_specs=[
                pl.BlockSpec((V, E), const),
                pl.BlockSpec((E, G), const),
                pl.BlockSpec((E, G), const),
                pl.BlockSpec((1, G), const),
                pl.BlockSpec((1, G), const),
                pl.BlockSpec((Hp, G), const),
                pl.BlockSpec((Hp, G), const),
                pl.BlockSpec((Hp, G), const),
                pl.BlockSpec((Hp, G), const),
                pl.BlockSpec((Hp, G), const),
                pl.BlockSpec((Hp, G), const),
                pl.BlockSpec((1, G), const),
                pl.BlockSpec((1, G), const),
                pl.BlockSpec((Hp, G), const),
                pl.BlockSpec((Hp, G), const),
                pl.BlockSpec((Hp, O), const),
                pl.BlockSpec((Hp, O), const),
                pl.BlockSpec((1, O), const),
            ],
            out_specs=pl.BlockSpec((B2, O), lambda c, p, t, tok: (c, 0)),
            scratch_shapes=[
                pltpu.VMEM((B2, 2 * Hp), jnp.float32),
                pltpu.VMEM((B2, 2 * Hp), jnp.float32),
                pltpu.VMEM((T * B2, E), jnp.float32),
                pltpu.VMEM((T * B2, Hp), jnp.bfloat16),
                pltpu.VMEM((T * B2, Hp), jnp.bfloat16),
                pltpu.VMEM((B2, O), jnp.float32),
                pltpu.VMEM((2 * Hp, 2 * G), jnp.bfloat16),
                pltpu.VMEM((2 * Hp, 2 * G), jnp.bfloat16),
                pltpu.VMEM((2 * Hp, G), jnp.bfloat16),
                pltpu.VMEM((2 * Hp, G), jnp.bfloat16),
                pltpu.VMEM((E, G), jnp.bfloat16),
                pltpu.VMEM((E, G), jnp.bfloat16),
            ],
        ),
        compiler_params=pltpu.CompilerParams(
            dimension_semantics=("parallel", "arbitrary", "arbitrary")),
    )(tokens.reshape(-1), embedding, l0_w_in_f0, l0_w_in_b0, l0_b_f, l0_b_b,
      l0_whh_f, l0_whh_b, l1_w_in_f0, l1_w_in_f1, l1_w_in_b0, l1_w_in_b1,
      l1_b_f, l1_b_b, l1_whh_f, l1_whh_b, fc_wf, fc_wb, fc_b)

    return out
